# baseline (speedup 1.0000x reference)
import functools

import jax
import jax.numpy as jnp
from jax import lax
from jax.experimental import pallas as pl
from jax.experimental.pallas import tpu as pltpu
from jax.experimental.pallas import tpu_sc as plsc

DT = 0.1
NC, NS, L = 2, 16, 16
NW = NC * NS
W = 128


def _build_hist(E, N_pad):
    nwin = E // W
    base_trips = nwin // NW
    extra = nwin % NW
    slc = N_pad // NS
    mesh = plsc.VectorSubcoreMesh(core_axis_name="c", subcore_axis_name="s")

    @functools.partial(
        pl.kernel,
        out_type=jax.ShapeDtypeStruct((NC, N_pad), jnp.float32),
        mesh=mesh,
        scratch_types=[
            pltpu.VMEM((W,), jnp.int32),
            pltpu.VMEM((W,), jnp.float32),
            pltpu.VMEM((slc,), jnp.float32),
            pltpu.VMEM_SHARED((N_pad,), jnp.float32),
        ],
    )
    def hist_kernel(src_hbm, deg_out, sv, ones, zbuf, hist):
        c = lax.axis_index("c")
        s = lax.axis_index("s")
        wid = s * NC + c

        def fill(i, _):
            ones[pl.ds(i * L, L)] = jnp.full((L,), 1.0, jnp.float32)
            return 0

        lax.fori_loop(0, W // L, fill, 0)

        def zfill(i, _):
            zbuf[pl.ds(i * L, L)] = jnp.zeros((L,), jnp.float32)
            return 0

        lax.fori_loop(0, slc // L, zfill, 0)
        pltpu.sync_copy(zbuf, hist.at[pl.ds(s * slc, slc)])
        plsc.subcore_barrier()

        ntrips = base_trips + jnp.where(wid < extra, 1, 0)

        def body(t, _):
            win = wid + t * NW
            pltpu.sync_copy(src_hbm.at[pl.ds(win * W, W)], sv)
            pltpu.sync_copy(ones, hist.at[sv], add=True)
            return 0

        lax.fori_loop(0, ntrips, body, 0)
        plsc.subcore_barrier()
        pltpu.sync_copy(hist.at[pl.ds(s * slc, slc)],
                        deg_out.at[c, pl.ds(s * slc, slc)])

    return hist_kernel


def _build_main(N, N_pad, Q, E):
    nwin = E // W
    base_trips = nwin // NW
    extra = nwin % NW
    slc = N_pad // NS
    rows_per_tile = N_pad // NS
    nfull = rows_per_tile // W
    rem = rows_per_tile % W
    mesh = plsc.VectorSubcoreMesh(core_axis_name="c", subcore_axis_name="s")

    @functools.partial(
        pl.kernel,
        out_type=(
            jax.ShapeDtypeStruct((NC, N_pad, Q), jnp.float32),
            jax.ShapeDtypeStruct((NC, N_pad), jnp.float32),
        ),
        mesh=mesh,
        scratch_types=[
            pltpu.VMEM((W,), jnp.int32),
            pltpu.VMEM((W,), jnp.int32),
            pltpu.VMEM((W,), jnp.float32),
            pltpu.VMEM((W,), jnp.float32),
            pltpu.VMEM((W,), jnp.float32),
            pltpu.VMEM((W, Q), jnp.float32),
            pltpu.VMEM((W, Q), jnp.float32),
            pltpu.VMEM((slc,), jnp.float32),
            pltpu.VMEM((slc,), jnp.float32),
            pltpu.VMEM((slc,), jnp.float32),
            pltpu.VMEM_SHARED((N_pad,), jnp.float32),
            pltpu.VMEM_SHARED((N_pad, Q), jnp.float32),
            pltpu.VMEM_SHARED((N_pad,), jnp.float32),
            pltpu.SemaphoreType.DMA,
            pltpu.SemaphoreType.DMA,
            pltpu.SemaphoreType.DMA,
            pltpu.SemaphoreType.DMA,
        ],
    )
    def main_kernel(f_hbm, src_hbm, dst_hbm, ew_hbm, degp_hbm,
                    t_out, s_out,
                    sv, dv, wv, cbuf, gbuf, bufD, bufS, da, db, invloc,
                    invdeg_sh, Tacc, Sacc, semi, sem1, sem2, semsc):
        c = lax.axis_index("c")
        s = lax.axis_index("s")
        wid = s * NC + c

        pltpu.sync_copy(degp_hbm.at[0, pl.ds(s * slc, slc)], da)
        pltpu.sync_copy(degp_hbm.at[1, pl.ds(s * slc, slc)], db)

        def mkinv(i, _):
            d = da[pl.ds(i * L, L)] + db[pl.ds(i * L, L)]
            invloc[pl.ds(i * L, L)] = 1.0 / jnp.maximum(d, 1.0)
            return 0

        lax.fori_loop(0, slc // L, mkinv, 0)
        pltpu.sync_copy(invloc, invdeg_sh.at[pl.ds(s * slc, slc)])

        def zrow(j, _):
            for kk in range(Q // L):
                bufD[j, pl.ds(kk * L, L)] = jnp.zeros((L,), jnp.float32)
            return 0

        lax.fori_loop(0, W, zrow, 0)
        base_row = s * rows_per_tile

        def zcp(i, _):
            pltpu.sync_copy(bufD, Tacc.at[pl.ds(base_row + i * W, W)])
            return 0

        lax.fori_loop(0, nfull, zcp, 0)
        if rem:
            pltpu.sync_copy(bufD.at[pl.ds(0, rem)],
                            Tacc.at[pl.ds(base_row + nfull * W, rem)])

        def zfill(i, _):
            cbuf[pl.ds(i * L, L)] = jnp.zeros((L,), jnp.float32)
            return 0

        lax.fori_loop(0, W // L, zfill, 0)
        nsc = slc // W
        rsc = slc % W

        def zsc(i, _):
            pltpu.sync_copy(cbuf, Sacc.at[pl.ds(s * slc + i * W, W)])
            return 0

        lax.fori_loop(0, nsc, zsc, 0)
        if rsc:
            pltpu.sync_copy(cbuf.at[pl.ds(0, rsc)],
                            Sacc.at[pl.ds(s * slc + nsc * W, rsc)])
        plsc.subcore_barrier()

        ntrips = base_trips + jnp.where(wid < extra, 1, 0)

        def body(t, _):
            win = wid + t * NW
            eb = win * W
            ii = [
                pltpu.async_copy(src_hbm.at[pl.ds(eb, W)], sv, semi),
                pltpu.async_copy(dst_hbm.at[pl.ds(eb, W)], dv, semi),
                pltpu.async_copy(ew_hbm.at[pl.ds(eb, W)], wv, semi),
            ]
            for d in ii:
                d.wait()
            gd = pltpu.async_copy(f_hbm.at[dv], bufD, sem1)
            gs = pltpu.async_copy(f_hbm.at[sv], bufS, sem2)
            gg = pltpu.async_copy(invdeg_sh.at[sv], gbuf, semi)

            gg.wait()

            @plsc.parallel_loop(0, W // L, unroll=2)
            def cgen(i):
                cbuf[pl.ds(i * L, L)] = (
                    wv[pl.ds(i * L, L)] * gbuf[pl.ds(i * L, L)])

            gd.wait()
            gs.wait()

            @plsc.parallel_loop(0, W // L, unroll=2)
            def scale(i):
                c16 = cbuf[pl.ds(i * L, L)]
                for ll in range(L):
                    cs = c16[ll]
                    j = i * L + ll
                    for kk in range(Q // L):
                        bufD[j, pl.ds(kk * L, L)] = (
                            bufD[j, pl.ds(kk * L, L)] * cs)
                        bufS[j, pl.ds(kk * L, L)] = (
                            bufS[j, pl.ds(kk * L, L)] * cs)

            ss = [
                pltpu.async_copy(bufD, Tacc.at[sv], semsc, add=True),
                pltpu.async_copy(bufS, Tacc.at[dv], semsc, add=True),
                pltpu.async_copy(cbuf, Sacc.at[sv], semsc, add=True),
                pltpu.async_copy(cbuf, Sacc.at[dv], semsc, add=True),
            ]
            for d in ss:
                d.wait()
            return 0

        lax.fori_loop(0, ntrips, body, 0)
        plsc.subcore_barrier()

        pltpu.sync_copy(Tacc.at[pl.ds(base_row, rows_per_tile)],
                        t_out.at[c, pl.ds(base_row, rows_per_tile)])
        pltpu.sync_copy(Sacc.at[pl.ds(s * slc, slc)],
                        s_out.at[c, pl.ds(s * slc, slc)])

    return main_kernel


def _build_clip(N, Q, blk):
    def body(f_ref, o_ref):
        o_ref[...] = jnp.maximum(f_ref[...], 0.0)

    return pl.pallas_call(
        body,
        grid=(N // blk,),
        in_specs=[pl.BlockSpec((blk, Q), lambda i: (i, 0))],
        out_specs=pl.BlockSpec((blk, Q), lambda i: (i, 0)),
        out_shape=jax.ShapeDtypeStruct((N, Q), jnp.float32),
    )


def _build_combine(N, N_pad, Q, blk):
    def body(fc_ref, co_ref, so_ref, t_ref, s0_ref, s1_ref, xi_ref, o_ref):
        fc = fc_ref[...]
        t = t_ref[0] + t_ref[1]
        svec = s0_ref[...] + s1_ref[...]
        transport = xi_ref[...] * (t - svec * fc)
        o_ref[...] = jnp.maximum(
            fc - DT * (transport - co_ref[...] - so_ref[...]), 0.0)

    return pl.pallas_call(
        body,
        grid=(N // blk,),
        in_specs=[
            pl.BlockSpec((blk, Q), lambda i: (i, 0)),
            pl.BlockSpec((blk, Q), lambda i: (i, 0)),
            pl.BlockSpec((blk, Q), lambda i: (i, 0)),
            pl.BlockSpec((NC, blk, Q), lambda i: (0, i, 0)),
            pl.BlockSpec((blk, 1), lambda i: (i, 0)),
            pl.BlockSpec((blk, 1), lambda i: (i, 0)),
            pl.BlockSpec((1, Q), lambda i: (0, 0)),
        ],
        out_specs=pl.BlockSpec((blk, Q), lambda i: (i, 0)),
        out_shape=jax.ShapeDtypeStruct((N, Q), jnp.float32),
    )


def kernel(f_distribution, collision_term, source_term, edge_index,
           edge_weight, xi_velocities):
    N, Q = f_distribution.shape
    E = edge_index.shape[1]
    N_pad = ((N + NS * L - 1) // (NS * L)) * (NS * L)
    src = edge_index[0]
    dst = edge_index[1]

    fc = _build_clip(N, Q, 1000)(f_distribution)
    degp = _build_hist(E, N_pad)(src)
    t_part, s_part = _build_main(N, N_pad, Q, E)(fc, src, dst, edge_weight,
                                                 degp)
    return _build_combine(N, N_pad, Q, 1000)(
        fc, collision_term, source_term, t_part,
        s_part[0, :N].reshape(N, 1), s_part[1, :N].reshape(N, 1),
        xi_velocities.reshape(1, Q))


# speedup vs baseline: 1.1177x; 1.1177x over previous
import functools

import jax
import jax.numpy as jnp
from jax import lax
from jax.experimental import pallas as pl
from jax.experimental.pallas import tpu as pltpu
from jax.experimental.pallas import tpu_sc as plsc

DT = 0.1
NC, NS, L = 2, 16, 16
NW = NC * NS
W = 128


def _build_hist(E, N_pad):
    nwin = E // W
    base_trips = nwin // NW
    extra = nwin % NW
    slc = N_pad // NS
    mesh = plsc.VectorSubcoreMesh(core_axis_name="c", subcore_axis_name="s")

    @functools.partial(
        pl.kernel,
        out_type=jax.ShapeDtypeStruct((NC, N_pad), jnp.float32),
        mesh=mesh,
        scratch_types=[
            pltpu.VMEM((W,), jnp.int32),
            pltpu.VMEM((W,), jnp.float32),
            pltpu.VMEM((slc,), jnp.float32),
            pltpu.VMEM_SHARED((N_pad,), jnp.float32),
        ],
    )
    def hist_kernel(src_hbm, deg_out, sv, ones, zbuf, hist):
        c = lax.axis_index("c")
        s = lax.axis_index("s")
        wid = s * NC + c

        def fill(i, _):
            ones[pl.ds(i * L, L)] = jnp.full((L,), 1.0, jnp.float32)
            return 0

        lax.fori_loop(0, W // L, fill, 0)

        def zfill(i, _):
            zbuf[pl.ds(i * L, L)] = jnp.zeros((L,), jnp.float32)
            return 0

        lax.fori_loop(0, slc // L, zfill, 0)
        pltpu.sync_copy(zbuf, hist.at[pl.ds(s * slc, slc)])
        plsc.subcore_barrier()

        ntrips = base_trips + jnp.where(wid < extra, 1, 0)

        def body(t, _):
            win = wid + t * NW
            pltpu.sync_copy(src_hbm.at[pl.ds(win * W, W)], sv)
            pltpu.sync_copy(ones, hist.at[sv], add=True)
            return 0

        lax.fori_loop(0, ntrips, body, 0)
        plsc.subcore_barrier()
        pltpu.sync_copy(hist.at[pl.ds(s * slc, slc)],
                        deg_out.at[c, pl.ds(s * slc, slc)])

    return hist_kernel


def _build_main(N, N_pad, Q, E):
    nwin = E // W
    base_trips = nwin // NW
    extra = nwin % NW
    slc = N_pad // NS
    rows_per_tile = N_pad // NS
    nfull = rows_per_tile // W
    rem = rows_per_tile % W
    mesh = plsc.VectorSubcoreMesh(core_axis_name="c", subcore_axis_name="s")

    @functools.partial(
        pl.kernel,
        out_type=(
            jax.ShapeDtypeStruct((NC, N_pad, Q), jnp.float32),
            jax.ShapeDtypeStruct((NC, N_pad), jnp.float32),
        ),
        mesh=mesh,
        scratch_types=[
            pltpu.VMEM((W,), jnp.int32),
            pltpu.VMEM((W,), jnp.int32),
            pltpu.VMEM((W,), jnp.float32),
            pltpu.VMEM((W,), jnp.float32),
            pltpu.VMEM((W,), jnp.float32),
            pltpu.VMEM((W, Q), jnp.float32),
            pltpu.VMEM((W, Q), jnp.float32),
            pltpu.VMEM((slc,), jnp.float32),
            pltpu.VMEM((slc,), jnp.float32),
            pltpu.VMEM((slc,), jnp.float32),
            pltpu.VMEM_SHARED((N_pad,), jnp.float32),
            pltpu.VMEM_SHARED((N_pad, Q), jnp.float32),
            pltpu.VMEM_SHARED((N_pad,), jnp.float32),
            pltpu.SemaphoreType.DMA,
            pltpu.SemaphoreType.DMA,
            pltpu.SemaphoreType.DMA,
            pltpu.SemaphoreType.DMA,
        ],
    )
    def main_kernel(f_hbm, src_hbm, dst_hbm, ew_hbm, degp_hbm,
                    t_out, s_out,
                    sv, dv, wv, cbuf, gbuf, bufD, bufS, da, db, invloc,
                    invdeg_sh, Tacc, Sacc, semi, sem1, sem2, semsc):
        c = lax.axis_index("c")
        s = lax.axis_index("s")
        wid = s * NC + c

        pltpu.sync_copy(degp_hbm.at[0, pl.ds(s * slc, slc)], da)
        pltpu.sync_copy(degp_hbm.at[1, pl.ds(s * slc, slc)], db)

        def mkinv(i, _):
            d = da[pl.ds(i * L, L)] + db[pl.ds(i * L, L)]
            invloc[pl.ds(i * L, L)] = 1.0 / jnp.maximum(d, 1.0)
            return 0

        lax.fori_loop(0, slc // L, mkinv, 0)
        pltpu.sync_copy(invloc, invdeg_sh.at[pl.ds(s * slc, slc)])

        def zrow(j, _):
            for kk in range(Q // L):
                bufD[j, pl.ds(kk * L, L)] = jnp.zeros((L,), jnp.float32)
            return 0

        lax.fori_loop(0, W, zrow, 0)
        base_row = s * rows_per_tile

        def zcp(i, _):
            pltpu.sync_copy(bufD, Tacc.at[pl.ds(base_row + i * W, W)])
            return 0

        lax.fori_loop(0, nfull, zcp, 0)
        if rem:
            pltpu.sync_copy(bufD.at[pl.ds(0, rem)],
                            Tacc.at[pl.ds(base_row + nfull * W, rem)])

        def zfill(i, _):
            cbuf[pl.ds(i * L, L)] = jnp.zeros((L,), jnp.float32)
            return 0

        lax.fori_loop(0, W // L, zfill, 0)
        nsc = slc // W
        rsc = slc % W

        def zsc(i, _):
            pltpu.sync_copy(cbuf, Sacc.at[pl.ds(s * slc + i * W, W)])
            return 0

        lax.fori_loop(0, nsc, zsc, 0)
        if rsc:
            pltpu.sync_copy(cbuf.at[pl.ds(0, rsc)],
                            Sacc.at[pl.ds(s * slc + nsc * W, rsc)])
        plsc.subcore_barrier()

        ntrips = base_trips + jnp.where(wid < extra, 1, 0)

        def body(t, _):
            win = wid + t * NW
            eb = win * W
            ii = [
                pltpu.async_copy(src_hbm.at[pl.ds(eb, W)], sv, semi),
                pltpu.async_copy(dst_hbm.at[pl.ds(eb, W)], dv, semi),
                pltpu.async_copy(ew_hbm.at[pl.ds(eb, W)], wv, semi),
            ]
            for d in ii:
                d.wait()
            gd = pltpu.async_copy(f_hbm.at[dv], bufD, sem1)
            gs = pltpu.async_copy(f_hbm.at[sv], bufS, sem2)
            gg = pltpu.async_copy(invdeg_sh.at[sv], gbuf, semi)

            gg.wait()

            def cgen(i, _):
                cbuf[pl.ds(i * L, L)] = (
                    wv[pl.ds(i * L, L)] * gbuf[pl.ds(i * L, L)])
                return 0

            lax.fori_loop(0, W // L, cgen, 0)
            gd.wait()
            gs.wait()

            def scale(i, _):
                c16 = cbuf[pl.ds(i * L, L)]
                for ll in range(L):
                    cs = c16[ll]
                    j = i * L + ll
                    for kk in range(Q // L):
                        bufD[j, pl.ds(kk * L, L)] = (
                            bufD[j, pl.ds(kk * L, L)] * cs)
                        bufS[j, pl.ds(kk * L, L)] = (
                            bufS[j, pl.ds(kk * L, L)] * cs)
                return 0

            lax.fori_loop(0, W // L, scale, 0)

            ss = [
                pltpu.async_copy(bufD, Tacc.at[sv], semsc, add=True),
                pltpu.async_copy(bufS, Tacc.at[dv], semsc, add=True),
                pltpu.async_copy(cbuf, Sacc.at[sv], semsc, add=True),
                pltpu.async_copy(cbuf, Sacc.at[dv], semsc, add=True),
            ]
            for d in ss:
                d.wait()
            return 0

        lax.fori_loop(0, ntrips, body, 0)
        plsc.subcore_barrier()

        pltpu.sync_copy(Tacc.at[pl.ds(base_row, rows_per_tile)],
                        t_out.at[c, pl.ds(base_row, rows_per_tile)])
        pltpu.sync_copy(Sacc.at[pl.ds(s * slc, slc)],
                        s_out.at[c, pl.ds(s * slc, slc)])

    return main_kernel


def _build_clip(N, Q, blk):
    def body(f_ref, o_ref):
        o_ref[...] = jnp.maximum(f_ref[...], 0.0)

    return pl.pallas_call(
        body,
        grid=(N // blk,),
        in_specs=[pl.BlockSpec((blk, Q), lambda i: (i, 0))],
        out_specs=pl.BlockSpec((blk, Q), lambda i: (i, 0)),
        out_shape=jax.ShapeDtypeStruct((N, Q), jnp.float32),
    )


def _build_combine(N, N_pad, Q, blk):
    def body(fc_ref, co_ref, so_ref, t_ref, s0_ref, s1_ref, xi_ref, o_ref):
        fc = fc_ref[...]
        t = t_ref[0] + t_ref[1]
        svec = s0_ref[...] + s1_ref[...]
        transport = xi_ref[...] * (t - svec * fc)
        o_ref[...] = jnp.maximum(
            fc - DT * (transport - co_ref[...] - so_ref[...]), 0.0)

    return pl.pallas_call(
        body,
        grid=(N // blk,),
        in_specs=[
            pl.BlockSpec((blk, Q), lambda i: (i, 0)),
            pl.BlockSpec((blk, Q), lambda i: (i, 0)),
            pl.BlockSpec((blk, Q), lambda i: (i, 0)),
            pl.BlockSpec((NC, blk, Q), lambda i: (0, i, 0)),
            pl.BlockSpec((blk, 1), lambda i: (i, 0)),
            pl.BlockSpec((blk, 1), lambda i: (i, 0)),
            pl.BlockSpec((1, Q), lambda i: (0, 0)),
        ],
        out_specs=pl.BlockSpec((blk, Q), lambda i: (i, 0)),
        out_shape=jax.ShapeDtypeStruct((N, Q), jnp.float32),
    )


def kernel(f_distribution, collision_term, source_term, edge_index,
           edge_weight, xi_velocities):
    N, Q = f_distribution.shape
    E = edge_index.shape[1]
    N_pad = ((N + NS * L - 1) // (NS * L)) * (NS * L)
    src = edge_index[0]
    dst = edge_index[1]

    fc = _build_clip(N, Q, 1000)(f_distribution)
    degp = _build_hist(E, N_pad)(src)
    t_part, s_part = _build_main(N, N_pad, Q, E)(fc, src, dst, edge_weight,
                                                 degp)
    return _build_combine(N, N_pad, Q, 1000)(
        fc, collision_term, source_term, t_part,
        s_part[0, :N].reshape(N, 1), s_part[1, :N].reshape(N, 1),
        xi_velocities.reshape(1, Q))
